# Initial kernel scaffold; baseline (speedup 1.0000x reference)
import functools

import jax
import jax.numpy as jnp
from jax import lax
from jax.experimental import pallas as pl
from jax.experimental.pallas import tpu as pltpu
from jax.experimental.pallas import tpu_sc as plsc

N = 10000
E = 320000
D_IN = 128
DIM = 256
H = DIM // 2
NC = 2
NS = 16
NPT = N // NS
B = 80
EPT = E // NS
NB = EPT // B
EPT2 = E // (NC * NS)
NB2 = EPT2 // B

_mesh = plsc.VectorSubcoreMesh(core_axis_name="c", subcore_axis_name="s")
_f32 = jnp.float32


def _deg_body(dst_hbm, out_hbm, di, ones, zrows, acc):
    c = lax.axis_index("c")
    t = lax.axis_index("s")

    def fill_ones(i, _):
        ones[i] = jnp.ones((16,), _f32)
        return 0

    lax.fori_loop(0, B, fill_ones, 0)

    def fill_zero(i, _):
        zrows[i] = jnp.zeros((16,), _f32)
        return 0

    lax.fori_loop(0, NPT, fill_zero, 0)
    pltpu.sync_copy(zrows, acc.at[pl.ds(t * NPT, NPT)])
    plsc.subcore_barrier()

    ebase = (c * NS + t) * EPT2

    def block(j, _):
        pltpu.sync_copy(dst_hbm.at[pl.ds(ebase + j * B, B)], di)
        pltpu.sync_copy(ones, acc.at[di], add=True)
        return 0

    lax.fori_loop(0, NB2, block, 0)
    plsc.subcore_barrier()
    pltpu.sync_copy(acc.at[pl.ds(t * NPT, NPT)],
                    out_hbm.at[pl.ds(c * N + t * NPT, NPT)])


_deg_call = pl.kernel(
    _deg_body,
    out_type=jax.ShapeDtypeStruct((2 * N, 16), _f32),
    mesh=_mesh,
    scratch_types=[
        pltpu.VMEM((B,), jnp.int32),
        pltpu.VMEM((B, 16), _f32),
        pltpu.VMEM((NPT, 16), _f32),
        pltpu.VMEM_SHARED((N, 16), _f32),
    ],
)


def _agg_body(z_hbm, src_hbm, dst_hbm, out_hbm, si, di, rows, acc, sem):
    c = lax.axis_index("c")
    t = lax.axis_index("s")
    nbase = t * NPT
    pltpu.sync_copy(z_hbm.at[pl.ds(c * N + nbase, NPT)],
                    acc.at[pl.ds(nbase, NPT)])
    plsc.subcore_barrier()

    ebase = t * EPT
    cN = c * N

    def block(j, _):
        off = ebase + j * B
        pltpu.sync_copy(src_hbm.at[pl.ds(off, B)], si)

        def adj(k, _):
            si[pl.ds(k * 16, 16)] = si[pl.ds(k * 16, 16)] + cN
            return 0

        lax.fori_loop(0, B // 16, adj, 0)
        pltpu.sync_copy(dst_hbm.at[pl.ds(off, B)], di)
        pltpu.async_copy(z_hbm.at[si], rows, sem).wait()
        pltpu.sync_copy(rows, acc.at[di], add=True)
        return 0

    lax.fori_loop(0, NB, block, 0)
    plsc.subcore_barrier()
    pltpu.sync_copy(acc.at[pl.ds(nbase, NPT)],
                    out_hbm.at[pl.ds(c * N + nbase, NPT)])


_agg_call = pl.kernel(
    _agg_body,
    out_type=jax.ShapeDtypeStruct((2 * N, H), _f32),
    mesh=_mesh,
    scratch_types=[
        pltpu.VMEM((B,), jnp.int32),
        pltpu.VMEM((B,), jnp.int32),
        pltpu.VMEM((B, H), _f32),
        pltpu.VMEM_SHARED((N, H), _f32),
        pltpu.SemaphoreType.DMA,
    ],
)


_PCHUNK = 624
_PB = 16


def _pool_body(h_hbm, batch_hbm, out_hbm, bi, rbuf, acc):
    c = lax.axis_index("c")
    t = lax.axis_index("s")

    def fz(i, _):
        def fz2(k, _):
            rbuf[i, pl.ds(k * 16, 16)] = jnp.zeros((16,), _f32)
            return 0

        lax.fori_loop(0, H // 16, fz2, 0)
        return 0

    lax.fori_loop(0, 4, fz, 0)
    pltpu.sync_copy(rbuf.at[pl.ds(0, 4)], acc.at[pl.ds(t * 4, 4)])
    plsc.subcore_barrier()

    base = t * _PCHUNK
    nb = jnp.where(t == NS - 1, 40, 39)

    def block(j, _):
        off = base + j * _PB
        pltpu.sync_copy(batch_hbm.at[pl.ds(off, _PB)], bi)
        pltpu.sync_copy(h_hbm.at[pl.ds(c * N + off, _PB)],
                        rbuf.at[pl.ds(0, _PB)])
        pltpu.sync_copy(rbuf.at[pl.ds(0, _PB)], acc.at[bi], add=True)
        return 0

    lax.fori_loop(0, nb, block, 0)
    plsc.subcore_barrier()

    @pl.when(t == 0)
    def _():
        pltpu.sync_copy(acc, out_hbm.at[pl.ds(c * 64, 64)])


_pool_call = pl.kernel(
    _pool_body,
    out_type=jax.ShapeDtypeStruct((2 * 64, H), _f32),
    mesh=_mesh,
    scratch_types=[
        pltpu.VMEM((_PB,), jnp.int32),
        pltpu.VMEM((_PB, H), _f32),
        pltpu.VMEM_SHARED((64, H), _f32),
    ],
)


_PREC = lax.Precision.HIGHEST


def _tc_pre_body(x_ref, w1_ref, dcnt_ref, z_ref, dis_ref):
    deg = dcnt_ref[0:N, 0:1] + dcnt_ref[N:2 * N, 0:1] + 1.0
    dis = lax.rsqrt(deg)
    y = jnp.dot(x_ref[...], w1_ref[...], preferred_element_type=_f32,
                precision=_PREC)
    z_ref[0:N, :] = dis * y[:, :H]
    z_ref[N:2 * N, :] = dis * y[:, H:]
    dis_ref[...] = dis


_tc_pre = pl.pallas_call(
    _tc_pre_body,
    out_shape=[
        jax.ShapeDtypeStruct((2 * N, H), _f32),
        jax.ShapeDtypeStruct((N, 1), _f32),
    ],
)


def _bn(u, g, beta):
    mu = jnp.mean(u, axis=0, keepdims=True)
    var = jnp.mean((u - mu) ** 2, axis=0, keepdims=True)
    return g * (u - mu) * lax.rsqrt(var + 1e-5) + beta


def _tc_mid_body(a_ref, dis_ref, b_ref, g_ref, beta_ref, w_ref, z_ref, *,
                 relu):
    dis = dis_ref[...]
    u0 = dis * a_ref[0:N, :] + b_ref[0:1, :H]
    u1 = dis * a_ref[N:2 * N, :] + b_ref[0:1, H:]
    if relu:
        u0 = jnp.maximum(u0, 0.0)
        u1 = jnp.maximum(u1, 0.0)
    h0 = _bn(u0, g_ref[0:1, :H], beta_ref[0:1, :H])
    h1 = _bn(u1, g_ref[0:1, H:], beta_ref[0:1, H:])
    w = w_ref[...]
    y = (jnp.dot(h0, w[:H, :], preferred_element_type=_f32, precision=_PREC)
         + jnp.dot(h1, w[H:, :], preferred_element_type=_f32,
                   precision=_PREC))
    z_ref[0:N, :] = dis * y[:, :H]
    z_ref[N:2 * N, :] = dis * y[:, H:]


def _make_tc_mid(relu):
    return pl.pallas_call(
        functools.partial(_tc_mid_body, relu=relu),
        out_shape=jax.ShapeDtypeStruct((2 * N, H), _f32),
    )


_tc_mid_relu = _make_tc_mid(True)
_tc_mid = _make_tc_mid(False)


def _tc_last_body(a_ref, dis_ref, b_ref, g_ref, beta_ref, h_ref):
    dis = dis_ref[...]
    u0 = dis * a_ref[0:N, :] + b_ref[0:1, :H]
    u1 = dis * a_ref[N:2 * N, :] + b_ref[0:1, H:]
    h_ref[0:N, :] = _bn(u0, g_ref[0:1, :H], beta_ref[0:1, :H])
    h_ref[N:2 * N, :] = _bn(u1, g_ref[0:1, H:], beta_ref[0:1, H:])


_tc_last = pl.pallas_call(
    _tc_last_body,
    out_shape=jax.ShapeDtypeStruct((2 * N, H), _f32),
)


def _tc_mlp_body(p_ref, wm_ref, bm_ref, gm_ref, betam_ref, wo_ref, bo_ref,
                 out_ref):
    p = jnp.concatenate([p_ref[0:64, :], p_ref[64:128, :]], axis=1)
    wm = wm_ref[...]
    bm = bm_ref[...]
    gm = gm_ref[...]
    betam = betam_ref[...]
    for i in range(2):
        p = jnp.maximum(
            jnp.dot(p, wm[i], preferred_element_type=_f32, precision=_PREC)
            + bm[i:i + 1, :], 0.0)
        p = _bn(p, gm[i:i + 1, :], betam[i:i + 1, :])
    out_ref[...] = (jnp.dot(p, wo_ref[...], preferred_element_type=_f32,
                            precision=_PREC) + bo_ref[0:1, :])


_tc_mlp = pl.pallas_call(
    _tc_mlp_body,
    out_shape=jax.ShapeDtypeStruct((64, 1), _f32),
)


def kernel(x, edge_index, batch, W1, b1, g1, beta1, Wh, bh, gh, betah,
           Wm, bm, gm, betam, Wo, bo):
    src = edge_index[0]
    dst = edge_index[1]
    dcnt = _deg_call(dst)
    z, dis = _tc_pre(x, W1, dcnt)
    a = _agg_call(z, src, dst)
    z = _tc_mid_relu(a, dis, b1.reshape(1, DIM), g1.reshape(1, DIM),
                     beta1.reshape(1, DIM), Wh[0])
    a = _agg_call(z, src, dst)
    z = _tc_mid(a, dis, bh[0].reshape(1, DIM), gh[0].reshape(1, DIM),
                betah[0].reshape(1, DIM), Wh[1])
    a = _agg_call(z, src, dst)
    z = _tc_mid(a, dis, bh[1].reshape(1, DIM), gh[1].reshape(1, DIM),
                betah[1].reshape(1, DIM), Wh[2])
    a = _agg_call(z, src, dst)
    h = _tc_last(a, dis, bh[2].reshape(1, DIM), gh[2].reshape(1, DIM),
                 betah[2].reshape(1, DIM))
    p = _pool_call(h, batch)
    out = _tc_mlp(p, Wm, bm, gm, betam, Wo, bo.reshape(1, 1))
    return out


# speedup vs baseline: 6.9414x; 6.9414x over previous
import functools

import jax
import jax.numpy as jnp
from jax import lax
from jax.experimental import pallas as pl
from jax.experimental.pallas import tpu as pltpu
from jax.experimental.pallas import tpu_sc as plsc

N = 10000
E = 320000
D_IN = 128
DIM = 256
H = DIM // 2
NC = 2
NS = 16
B = 80
EPT = E // NS
NB = EPT // B
EPT2 = E // (NC * NS)
NB2 = EPT2 // B
NROWCH = N // B

_mesh = plsc.VectorSubcoreMesh(core_axis_name="c", subcore_axis_name="s")
_f32 = jnp.float32
_i32 = jnp.int32


def _row_chunks(t):
    nch = jnp.where(t < 13, 8, 7)
    first = t * 8 - jnp.maximum(t - 13, 0)
    return nch, first


def _fill_idx(idx, base):
    def f(k, _):
        idx[pl.ds(k * 16, 16)] = jnp.arange(16, dtype=_i32) + (base + k * 16)
        return 0

    lax.fori_loop(0, B // 16, f, 0)


def _deg_body(dst_hbm, out_hbm, di, ones, zrows, acc, sem):
    c = lax.axis_index("c")
    t = lax.axis_index("s")

    def fill_ones(i, _):
        ones[i, pl.ds(0, 16)] = jnp.ones((16,), _f32)
        return 0

    lax.fori_loop(0, B, fill_ones, 0)

    def fill_zero(i, _):
        zrows[i, pl.ds(0, 16)] = jnp.zeros((16,), _f32)
        return 0

    lax.fori_loop(0, B, fill_zero, 0)

    nch, first = _row_chunks(t)

    def zchunk(k, _):
        _fill_idx(di, (first + k) * B)
        pltpu.sync_copy(zrows, acc.at[di])
        return 0

    lax.fori_loop(0, nch, zchunk, 0)
    plsc.subcore_barrier()

    ebase = (c * NS + t) * EPT2

    def block(j, _):
        pltpu.sync_copy(dst_hbm.at[pl.ds(ebase + j * B, B)], di)
        pltpu.sync_copy(ones, acc.at[di], add=True)
        return 0

    lax.fori_loop(0, NB2, block, 0)
    plsc.subcore_barrier()

    def rchunk(k, _):
        base = (first + k) * B
        _fill_idx(di, base)
        pltpu.async_copy(acc.at[di], zrows, sem).wait()
        pltpu.sync_copy(zrows, out_hbm.at[pl.ds(c * N + base, B)])
        return 0

    lax.fori_loop(0, nch, rchunk, 0)


_deg_call = pl.kernel(
    _deg_body,
    out_type=jax.ShapeDtypeStruct((2 * N, 16), _f32),
    mesh=_mesh,
    scratch_types=[
        pltpu.VMEM((B,), _i32),
        pltpu.VMEM((B, 16), _f32),
        pltpu.VMEM((B, 16), _f32),
        pltpu.VMEM_SHARED((N, 16), _f32),
        pltpu.SemaphoreType.DMA,
    ],
)


def _agg_body(z_hbm, src_hbm, dst_hbm, out_hbm, si, di, rows, acc, sem):
    c = lax.axis_index("c")
    t = lax.axis_index("s")
    cN = c * N
    nch, first = _row_chunks(t)

    def zchunk(k, _):
        base = (first + k) * B
        pltpu.sync_copy(z_hbm.at[pl.ds(cN + base, B)], rows)
        _fill_idx(di, base)
        pltpu.sync_copy(rows, acc.at[di])
        return 0

    lax.fori_loop(0, nch, zchunk, 0)
    plsc.subcore_barrier()

    ebase = t * EPT

    def block(j, _):
        off = ebase + j * B
        pltpu.sync_copy(src_hbm.at[pl.ds(off, B)], si)

        def adj(k, _):
            si[pl.ds(k * 16, 16)] = si[pl.ds(k * 16, 16)] + cN
            return 0

        lax.fori_loop(0, B // 16, adj, 0)
        pltpu.sync_copy(dst_hbm.at[pl.ds(off, B)], di)
        pltpu.async_copy(z_hbm.at[si], rows, sem).wait()
        pltpu.sync_copy(rows, acc.at[di], add=True)
        return 0

    lax.fori_loop(0, NB, block, 0)
    plsc.subcore_barrier()

    def rchunk(k, _):
        base = (first + k) * B
        _fill_idx(di, base)
        pltpu.async_copy(acc.at[di], rows, sem).wait()
        pltpu.sync_copy(rows, out_hbm.at[pl.ds(cN + base, B)])
        return 0

    lax.fori_loop(0, nch, rchunk, 0)


_agg_call = pl.kernel(
    _agg_body,
    out_type=jax.ShapeDtypeStruct((2 * N, H), _f32),
    mesh=_mesh,
    scratch_types=[
        pltpu.VMEM((B,), _i32),
        pltpu.VMEM((B,), _i32),
        pltpu.VMEM((B, H), _f32),
        pltpu.VMEM_SHARED((N, H), _f32),
        pltpu.SemaphoreType.DMA,
    ],
)


_PB = 16


def _pool_body(h_hbm, batch_hbm, out_hbm, bi, gi, rload, rbuf, acc, sem):
    c = lax.axis_index("c")
    t = lax.axis_index("s")

    def fz(i, _):
        def fz2(k, _):
            rbuf[i, pl.ds(k * 16, 16)] = jnp.zeros((16,), _f32)
            return 0

        lax.fori_loop(0, H // 16, fz2, 0)
        return 0

    lax.fori_loop(0, _PB, fz, 0)

    @pl.when(t < 4)
    def _():
        gi[pl.ds(0, 16)] = jnp.arange(16, dtype=_i32) + t * 16
        pltpu.sync_copy(rbuf, acc.at[gi])

    plsc.subcore_barrier()

    base = t * 624
    nb = jnp.where(t == NS - 1, 40, 39)

    def block(j, _):
        off = base + j * _PB
        pltpu.sync_copy(batch_hbm.at[pl.ds(off, _PB)], bi)
        pltpu.sync_copy(h_hbm.at[pl.ds(c * N + off, _PB)], rload)
        pltpu.sync_copy(rload, acc.at[bi], add=True)
        return 0

    lax.fori_loop(0, nb, block, 0)
    plsc.subcore_barrier()

    @pl.when(t < 4)
    def _():
        pltpu.async_copy(acc.at[gi], rbuf, sem).wait()
        pltpu.sync_copy(rbuf, out_hbm.at[pl.ds(c * 64 + t * 16, 16)])


_pool_call = pl.kernel(
    _pool_body,
    out_type=jax.ShapeDtypeStruct((2 * 64, H), _f32),
    mesh=_mesh,
    scratch_types=[
        pltpu.VMEM((_PB,), _i32),
        pltpu.VMEM((_PB,), _i32),
        pltpu.VMEM((_PB, H), _f32),
        pltpu.VMEM((_PB, H), _f32),
        pltpu.VMEM_SHARED((64, H), _f32),
        pltpu.SemaphoreType.DMA,
    ],
)


_PREC = lax.Precision.HIGHEST
_EPS = 1e-5
_R = 5
_BR = N // _R


def _dot(a, b):
    return jnp.dot(a, b, preferred_element_type=_f32, precision=_PREC)


def _tc_pre_body(x_ref, w1_ref, dcnt_ref, z_ref, dis_ref):
    deg = dcnt_ref[0, :, 0:1] + dcnt_ref[1, :, 0:1] + 1.0
    dis = lax.rsqrt(deg)
    y = _dot(x_ref[...], w1_ref[...])
    z_ref[0] = dis * y[:, :H]
    z_ref[1] = dis * y[:, H:]
    dis_ref[...] = dis


_tc_pre = pl.pallas_call(
    _tc_pre_body,
    grid=(_R,),
    in_specs=[
        pl.BlockSpec((_BR, D_IN), lambda r: (r, 0)),
        pl.BlockSpec((D_IN, DIM), lambda r: (0, 0)),
        pl.BlockSpec((2, _BR, 16), lambda r: (0, r, 0)),
    ],
    out_specs=[
        pl.BlockSpec((2, _BR, H), lambda r: (0, r, 0)),
        pl.BlockSpec((_BR, 1), lambda r: (r, 0)),
    ],
    out_shape=[
        jax.ShapeDtypeStruct((2, N, H), _f32),
        jax.ShapeDtypeStruct((N, 1), _f32),
    ],
)


def _u_halves(a_ref, dis_ref, b_ref, relu):
    dis = dis_ref[...]
    u0 = dis * a_ref[0] + b_ref[0:1, :H]
    u1 = dis * a_ref[1] + b_ref[0:1, H:]
    if relu:
        u0 = jnp.maximum(u0, 0.0)
        u1 = jnp.maximum(u1, 0.0)
    return dis, u0, u1


def _stats_phase(p, r, sacc, u0, u1):
    @pl.when((p == 0) & (r == 0))
    def _():
        sacc[...] = jnp.zeros_like(sacc)

    @pl.when(p == 0)
    def _():
        sacc[0:1, :] += jnp.sum(u0, axis=0, keepdims=True)
        sacc[1:2, :] += jnp.sum(u1, axis=0, keepdims=True)
        sacc[2:3, :] += jnp.sum(u0 * u0, axis=0, keepdims=True)
        sacc[3:4, :] += jnp.sum(u1 * u1, axis=0, keepdims=True)


def _bn_apply(sacc, u0, u1, g_ref, beta_ref):
    s = sacc[...]
    inv_n = 1.0 / N
    mu0 = s[0:1, :] * inv_n
    mu1 = s[1:2, :] * inv_n
    v0 = s[2:3, :] * inv_n - mu0 * mu0
    v1 = s[3:4, :] * inv_n - mu1 * mu1
    h0 = g_ref[0:1, :H] * (u0 - mu0) * lax.rsqrt(v0 + _EPS) + beta_ref[0:1, :H]
    h1 = g_ref[0:1, H:] * (u1 - mu1) * lax.rsqrt(v1 + _EPS) + beta_ref[0:1, H:]
    return h0, h1


def _tc_mid_body(a_ref, dis_ref, b_ref, g_ref, beta_ref, w_ref, z_ref, sacc,
                 *, relu):
    p = pl.program_id(0)
    r = pl.program_id(1)
    dis, u0, u1 = _u_halves(a_ref, dis_ref, b_ref, relu)
    _stats_phase(p, r, sacc, u0, u1)

    @pl.when(p == 0)
    def _():
        z_ref[0] = u0
        z_ref[1] = u1

    @pl.when(p == 1)
    def _():
        h0, h1 = _bn_apply(sacc, u0, u1, g_ref, beta_ref)
        w = w_ref[...]
        y = _dot(h0, w[:H, :]) + _dot(h1, w[H:, :])
        z_ref[0] = dis * y[:, :H]
        z_ref[1] = dis * y[:, H:]


def _make_tc_mid(relu):
    return pl.pallas_call(
        functools.partial(_tc_mid_body, relu=relu),
        grid=(2, _R),
        in_specs=[
            pl.BlockSpec((2, _BR, H), lambda p, r: (0, r, 0)),
            pl.BlockSpec((_BR, 1), lambda p, r: (r, 0)),
            pl.BlockSpec((1, DIM), lambda p, r: (0, 0)),
            pl.BlockSpec((1, DIM), lambda p, r: (0, 0)),
            pl.BlockSpec((1, DIM), lambda p, r: (0, 0)),
            pl.BlockSpec((DIM, DIM), lambda p, r: (0, 0)),
        ],
        out_specs=pl.BlockSpec((2, _BR, H), lambda p, r: (0, r, 0)),
        out_shape=jax.ShapeDtypeStruct((2, N, H), _f32),
        scratch_shapes=[pltpu.VMEM((8, H), _f32)],
    )


_tc_mid_relu = _make_tc_mid(True)
_tc_mid = _make_tc_mid(False)


def _tc_last_body(a_ref, dis_ref, b_ref, g_ref, beta_ref, h_ref, sacc):
    p = pl.program_id(0)
    r = pl.program_id(1)
    _, u0, u1 = _u_halves(a_ref, dis_ref, b_ref, False)
    _stats_phase(p, r, sacc, u0, u1)

    @pl.when(p == 0)
    def _():
        h_ref[0] = u0
        h_ref[1] = u1

    @pl.when(p == 1)
    def _():
        h0, h1 = _bn_apply(sacc, u0, u1, g_ref, beta_ref)
        h_ref[0] = h0
        h_ref[1] = h1


_tc_last = pl.pallas_call(
    _tc_last_body,
    grid=(2, _R),
    in_specs=[
        pl.BlockSpec((2, _BR, H), lambda p, r: (0, r, 0)),
        pl.BlockSpec((_BR, 1), lambda p, r: (r, 0)),
        pl.BlockSpec((1, DIM), lambda p, r: (0, 0)),
        pl.BlockSpec((1, DIM), lambda p, r: (0, 0)),
        pl.BlockSpec((1, DIM), lambda p, r: (0, 0)),
    ],
    out_specs=pl.BlockSpec((2, _BR, H), lambda p, r: (0, r, 0)),
    out_shape=jax.ShapeDtypeStruct((2, N, H), _f32),
    scratch_shapes=[pltpu.VMEM((8, H), _f32)],
)


def _bn(u, g, beta):
    mu = jnp.mean(u, axis=0, keepdims=True)
    var = jnp.mean((u - mu) ** 2, axis=0, keepdims=True)
    return g * (u - mu) * lax.rsqrt(var + _EPS) + beta


def _tc_mlp_body(p_ref, wm0_ref, wm1_ref, bm0_ref, bm1_ref, gm0_ref,
                 gm1_ref, betam0_ref, betam1_ref, wo_ref, bo_ref, out_ref):
    p = jnp.concatenate([p_ref[0:64, :], p_ref[64:128, :]], axis=1)
    for wm_ref, bm_ref, gm_ref, betam_ref in (
            (wm0_ref, bm0_ref, gm0_ref, betam0_ref),
            (wm1_ref, bm1_ref, gm1_ref, betam1_ref)):
        p = jnp.maximum(_dot(p, wm_ref[...]) + bm_ref[0:1, :], 0.0)
        p = _bn(p, gm_ref[0:1, :], betam_ref[0:1, :])
    out_ref[...] = _dot(p, wo_ref[...]) + bo_ref[0:1, :]


_tc_mlp = pl.pallas_call(
    _tc_mlp_body,
    out_shape=jax.ShapeDtypeStruct((64, 1), _f32),
)


def kernel(x, edge_index, batch, W1, b1, g1, beta1, Wh, bh, gh, betah,
           Wm, bm, gm, betam, Wo, bo):
    src = edge_index[0]
    dst = edge_index[1]
    dcnt = _deg_call(dst)
    z3, dis = _tc_pre(x, W1, dcnt.reshape(2, N, 16))
    a = _agg_call(z3.reshape(2 * N, H), src, dst)
    z3 = _tc_mid_relu(a.reshape(2, N, H), dis, b1.reshape(1, DIM),
                      g1.reshape(1, DIM), beta1.reshape(1, DIM), Wh[0])
    a = _agg_call(z3.reshape(2 * N, H), src, dst)
    z3 = _tc_mid(a.reshape(2, N, H), dis, bh[0].reshape(1, DIM),
                 gh[0].reshape(1, DIM), betah[0].reshape(1, DIM), Wh[1])
    a = _agg_call(z3.reshape(2 * N, H), src, dst)
    z3 = _tc_mid(a.reshape(2, N, H), dis, bh[1].reshape(1, DIM),
                 gh[1].reshape(1, DIM), betah[1].reshape(1, DIM), Wh[2])
    a = _agg_call(z3.reshape(2 * N, H), src, dst)
    h3 = _tc_last(a.reshape(2, N, H), dis, bh[2].reshape(1, DIM),
                  gh[2].reshape(1, DIM), betah[2].reshape(1, DIM))
    p2 = _pool_call(h3.reshape(2 * N, H), batch)
    out = _tc_mlp(p2, Wm[0], Wm[1], bm[0:1], bm[1:2], gm[0:1], gm[1:2],
                  betam[0:1], betam[1:2], Wo, bo.reshape(1, 1))
    return out


# speedup vs baseline: 10.8947x; 1.5695x over previous
import functools

import jax
import jax.numpy as jnp
from jax import lax
from jax.experimental import pallas as pl
from jax.experimental.pallas import tpu as pltpu
from jax.experimental.pallas import tpu_sc as plsc

N = 10000
E = 320000
D_IN = 128
DIM = 256
H = DIM // 2
NC = 2
NS = 16
B = 80
EPT = E // NS
NB = EPT // B
EPT2 = E // (NC * NS)
NB2 = EPT2 // B
NROWCH = N // B

_mesh = plsc.VectorSubcoreMesh(core_axis_name="c", subcore_axis_name="s")
_f32 = jnp.float32
_i32 = jnp.int32


def _row_chunks(t):
    nch = jnp.where(t < 13, 8, 7)
    first = t * 8 - jnp.maximum(t - 13, 0)
    return nch, first


def _fill_idx(idx, base):
    def f(k, _):
        idx[pl.ds(k * 16, 16)] = jnp.arange(16, dtype=_i32) + (base + k * 16)
        return 0

    lax.fori_loop(0, B // 16, f, 0)


def _deg_body(dst_hbm, out_hbm, di, ones, zrows, acc, sem):
    c = lax.axis_index("c")
    t = lax.axis_index("s")

    def fill_ones(i, _):
        ones[i, pl.ds(0, 16)] = jnp.ones((16,), _f32)
        return 0

    lax.fori_loop(0, B, fill_ones, 0)

    def fill_zero(i, _):
        zrows[i, pl.ds(0, 16)] = jnp.zeros((16,), _f32)
        return 0

    lax.fori_loop(0, B, fill_zero, 0)

    nch, first = _row_chunks(t)

    def zchunk(k, _):
        _fill_idx(di, (first + k) * B)
        pltpu.sync_copy(zrows, acc.at[di])
        return 0

    lax.fori_loop(0, nch, zchunk, 0)
    plsc.subcore_barrier()

    ebase = (c * NS + t) * EPT2

    def block(j, _):
        pltpu.sync_copy(dst_hbm.at[pl.ds(ebase + j * B, B)], di)
        pltpu.sync_copy(ones, acc.at[di], add=True)
        return 0

    lax.fori_loop(0, NB2, block, 0)
    plsc.subcore_barrier()

    def rchunk(k, _):
        base = (first + k) * B
        _fill_idx(di, base)
        pltpu.async_copy(acc.at[di], zrows, sem).wait()
        pltpu.sync_copy(zrows, out_hbm.at[pl.ds(c * N + base, B)])
        return 0

    lax.fori_loop(0, nch, rchunk, 0)


_deg_call = pl.kernel(
    _deg_body,
    out_type=jax.ShapeDtypeStruct((2 * N, 16), _f32),
    mesh=_mesh,
    scratch_types=[
        pltpu.VMEM((B,), _i32),
        pltpu.VMEM((B, 16), _f32),
        pltpu.VMEM((B, 16), _f32),
        pltpu.VMEM_SHARED((N, 16), _f32),
        pltpu.SemaphoreType.DMA,
    ],
)


def _agg_body(z_hbm, src_hbm, dst_hbm, out_hbm, si, di, rows, si2, di2,
              rows2, acc, sem, semg, semg2, sems, sems2):
    c = lax.axis_index("c")
    t = lax.axis_index("s")
    cN = c * N
    nch, first = _row_chunks(t)

    def zchunk(k, _):
        base = (first + k) * B
        pltpu.sync_copy(z_hbm.at[pl.ds(cN + base, B)], rows)
        _fill_idx(di, base)
        pltpu.sync_copy(rows, acc.at[di])
        return 0

    lax.fori_loop(0, nch, zchunk, 0)
    plsc.subcore_barrier()

    ebase = t * EPT

    def pair(j, _):
        offa = ebase + 2 * j * B
        offb = offa + B
        d1 = pltpu.async_copy(src_hbm.at[pl.ds(offa, B)], si, sem)
        d2 = pltpu.async_copy(dst_hbm.at[pl.ds(offa, B)], di, sem)
        d3 = pltpu.async_copy(src_hbm.at[pl.ds(offb, B)], si2, sem)
        d4 = pltpu.async_copy(dst_hbm.at[pl.ds(offb, B)], di2, sem)
        d1.wait()
        d2.wait()
        d3.wait()
        d4.wait()

        def adj(k, _):
            si[pl.ds(k * 16, 16)] = si[pl.ds(k * 16, 16)] + cN
            si2[pl.ds(k * 16, 16)] = si2[pl.ds(k * 16, 16)] + cN
            return 0

        lax.fori_loop(0, B // 16, adj, 0)
        ga = pltpu.async_copy(z_hbm.at[si], rows, semg)
        gb = pltpu.async_copy(z_hbm.at[si2], rows2, semg2)
        ga.wait()
        sa = pltpu.async_copy(rows, acc.at[di], sems, add=True)
        gb.wait()
        sb = pltpu.async_copy(rows2, acc.at[di2], sems2, add=True)
        sa.wait()
        sb.wait()
        return 0

    lax.fori_loop(0, NB // 2, pair, 0)
    plsc.subcore_barrier()

    def rchunk(k, _):
        base = (first + k) * B
        _fill_idx(di, base)
        pltpu.async_copy(acc.at[di], rows, semg).wait()
        pltpu.sync_copy(rows, out_hbm.at[pl.ds(cN + base, B)])
        return 0

    lax.fori_loop(0, nch, rchunk, 0)


_agg_call = pl.kernel(
    _agg_body,
    out_type=jax.ShapeDtypeStruct((2 * N, H), _f32),
    mesh=_mesh,
    scratch_types=[
        pltpu.VMEM((B,), _i32),
        pltpu.VMEM((B,), _i32),
        pltpu.VMEM((B, H), _f32),
        pltpu.VMEM((B,), _i32),
        pltpu.VMEM((B,), _i32),
        pltpu.VMEM((B, H), _f32),
        pltpu.VMEM_SHARED((N, H), _f32),
        pltpu.SemaphoreType.DMA,
        pltpu.SemaphoreType.DMA,
        pltpu.SemaphoreType.DMA,
        pltpu.SemaphoreType.DMA,
        pltpu.SemaphoreType.DMA,
    ],
)


_PB = 16


def _pool_body(h_hbm, batch_hbm, out_hbm, bi, gi, rload, rbuf, acc, sem):
    c = lax.axis_index("c")
    t = lax.axis_index("s")

    def fz(i, _):
        def fz2(k, _):
            rbuf[i, pl.ds(k * 16, 16)] = jnp.zeros((16,), _f32)
            return 0

        lax.fori_loop(0, H // 16, fz2, 0)
        return 0

    lax.fori_loop(0, _PB, fz, 0)

    @pl.when(t < 4)
    def _():
        gi[pl.ds(0, 16)] = jnp.arange(16, dtype=_i32) + t * 16
        pltpu.sync_copy(rbuf, acc.at[gi])

    plsc.subcore_barrier()

    base = t * 624
    nb = jnp.where(t == NS - 1, 40, 39)

    def block(j, _):
        off = base + j * _PB
        pltpu.sync_copy(batch_hbm.at[pl.ds(off, _PB)], bi)
        pltpu.sync_copy(h_hbm.at[pl.ds(c * N + off, _PB)], rload)
        pltpu.sync_copy(rload, acc.at[bi], add=True)
        return 0

    lax.fori_loop(0, nb, block, 0)
    plsc.subcore_barrier()

    @pl.when(t < 4)
    def _():
        pltpu.async_copy(acc.at[gi], rbuf, sem).wait()
        pltpu.sync_copy(rbuf, out_hbm.at[pl.ds(c * 64 + t * 16, 16)])


_pool_call = pl.kernel(
    _pool_body,
    out_type=jax.ShapeDtypeStruct((2 * 64, H), _f32),
    mesh=_mesh,
    scratch_types=[
        pltpu.VMEM((_PB,), _i32),
        pltpu.VMEM((_PB,), _i32),
        pltpu.VMEM((_PB, H), _f32),
        pltpu.VMEM((_PB, H), _f32),
        pltpu.VMEM_SHARED((64, H), _f32),
        pltpu.SemaphoreType.DMA,
    ],
)


_PREC = lax.Precision.HIGHEST
_EPS = 1e-5
_R = 5
_BR = N // _R


def _dot(a, b):
    return jnp.dot(a, b, preferred_element_type=_f32, precision=_PREC)


def _tc_pre_body(x_ref, w1_ref, dcnt_ref, z_ref, dis_ref):
    deg = dcnt_ref[0, :, 0:1] + dcnt_ref[1, :, 0:1] + 1.0
    dis = lax.rsqrt(deg)
    y = _dot(x_ref[...], w1_ref[...])
    z_ref[0] = dis * y[:, :H]
    z_ref[1] = dis * y[:, H:]
    dis_ref[...] = dis


_tc_pre = pl.pallas_call(
    _tc_pre_body,
    grid=(_R,),
    in_specs=[
        pl.BlockSpec((_BR, D_IN), lambda r: (r, 0)),
        pl.BlockSpec((D_IN, DIM), lambda r: (0, 0)),
        pl.BlockSpec((2, _BR, 16), lambda r: (0, r, 0)),
    ],
    out_specs=[
        pl.BlockSpec((2, _BR, H), lambda r: (0, r, 0)),
        pl.BlockSpec((_BR, 1), lambda r: (r, 0)),
    ],
    out_shape=[
        jax.ShapeDtypeStruct((2, N, H), _f32),
        jax.ShapeDtypeStruct((N, 1), _f32),
    ],
)


def _u_halves(a_ref, dis_ref, b_ref, relu):
    dis = dis_ref[...]
    u0 = dis * a_ref[0] + b_ref[0:1, :H]
    u1 = dis * a_ref[1] + b_ref[0:1, H:]
    if relu:
        u0 = jnp.maximum(u0, 0.0)
        u1 = jnp.maximum(u1, 0.0)
    return dis, u0, u1


def _stats_phase(p, r, sacc, u0, u1):
    @pl.when((p == 0) & (r == 0))
    def _():
        sacc[...] = jnp.zeros_like(sacc)

    @pl.when(p == 0)
    def _():
        sacc[0:1, :] += jnp.sum(u0, axis=0, keepdims=True)
        sacc[1:2, :] += jnp.sum(u1, axis=0, keepdims=True)
        sacc[2:3, :] += jnp.sum(u0 * u0, axis=0, keepdims=True)
        sacc[3:4, :] += jnp.sum(u1 * u1, axis=0, keepdims=True)


def _bn_apply(sacc, u0, u1, g_ref, beta_ref):
    s = sacc[...]
    inv_n = 1.0 / N
    mu0 = s[0:1, :] * inv_n
    mu1 = s[1:2, :] * inv_n
    v0 = s[2:3, :] * inv_n - mu0 * mu0
    v1 = s[3:4, :] * inv_n - mu1 * mu1
    h0 = g_ref[0:1, :H] * (u0 - mu0) * lax.rsqrt(v0 + _EPS) + beta_ref[0:1, :H]
    h1 = g_ref[0:1, H:] * (u1 - mu1) * lax.rsqrt(v1 + _EPS) + beta_ref[0:1, H:]
    return h0, h1


def _tc_mid_body(a_ref, dis_ref, b_ref, g_ref, beta_ref, w_ref, z_ref, sacc,
                 *, relu):
    p = pl.program_id(0)
    r = pl.program_id(1)
    dis, u0, u1 = _u_halves(a_ref, dis_ref, b_ref, relu)
    _stats_phase(p, r, sacc, u0, u1)

    @pl.when(p == 0)
    def _():
        z_ref[0] = u0
        z_ref[1] = u1

    @pl.when(p == 1)
    def _():
        h0, h1 = _bn_apply(sacc, u0, u1, g_ref, beta_ref)
        w = w_ref[...]
        y = _dot(h0, w[:H, :]) + _dot(h1, w[H:, :])
        z_ref[0] = dis * y[:, :H]
        z_ref[1] = dis * y[:, H:]


def _make_tc_mid(relu):
    return pl.pallas_call(
        functools.partial(_tc_mid_body, relu=relu),
        grid=(2, _R),
        in_specs=[
            pl.BlockSpec((2, _BR, H), lambda p, r: (0, r, 0)),
            pl.BlockSpec((_BR, 1), lambda p, r: (r, 0)),
            pl.BlockSpec((1, DIM), lambda p, r: (0, 0)),
            pl.BlockSpec((1, DIM), lambda p, r: (0, 0)),
            pl.BlockSpec((1, DIM), lambda p, r: (0, 0)),
            pl.BlockSpec((DIM, DIM), lambda p, r: (0, 0)),
        ],
        out_specs=pl.BlockSpec((2, _BR, H), lambda p, r: (0, r, 0)),
        out_shape=jax.ShapeDtypeStruct((2, N, H), _f32),
        scratch_shapes=[pltpu.VMEM((8, H), _f32)],
    )


_tc_mid_relu = _make_tc_mid(True)
_tc_mid = _make_tc_mid(False)


def _tc_last_body(a_ref, dis_ref, b_ref, g_ref, beta_ref, h_ref, sacc):
    p = pl.program_id(0)
    r = pl.program_id(1)
    _, u0, u1 = _u_halves(a_ref, dis_ref, b_ref, False)
    _stats_phase(p, r, sacc, u0, u1)

    @pl.when(p == 0)
    def _():
        h_ref[0] = u0
        h_ref[1] = u1

    @pl.when(p == 1)
    def _():
        h0, h1 = _bn_apply(sacc, u0, u1, g_ref, beta_ref)
        h_ref[0] = h0
        h_ref[1] = h1


_tc_last = pl.pallas_call(
    _tc_last_body,
    grid=(2, _R),
    in_specs=[
        pl.BlockSpec((2, _BR, H), lambda p, r: (0, r, 0)),
        pl.BlockSpec((_BR, 1), lambda p, r: (r, 0)),
        pl.BlockSpec((1, DIM), lambda p, r: (0, 0)),
        pl.BlockSpec((1, DIM), lambda p, r: (0, 0)),
        pl.BlockSpec((1, DIM), lambda p, r: (0, 0)),
    ],
    out_specs=pl.BlockSpec((2, _BR, H), lambda p, r: (0, r, 0)),
    out_shape=jax.ShapeDtypeStruct((2, N, H), _f32),
    scratch_shapes=[pltpu.VMEM((8, H), _f32)],
)


def _bn(u, g, beta):
    mu = jnp.mean(u, axis=0, keepdims=True)
    var = jnp.mean((u - mu) ** 2, axis=0, keepdims=True)
    return g * (u - mu) * lax.rsqrt(var + _EPS) + beta


def _tc_mlp_body(p_ref, wm0_ref, wm1_ref, bm0_ref, bm1_ref, gm0_ref,
                 gm1_ref, betam0_ref, betam1_ref, wo_ref, bo_ref, out_ref):
    p = jnp.concatenate([p_ref[0:64, :], p_ref[64:128, :]], axis=1)
    for wm_ref, bm_ref, gm_ref, betam_ref in (
            (wm0_ref, bm0_ref, gm0_ref, betam0_ref),
            (wm1_ref, bm1_ref, gm1_ref, betam1_ref)):
        p = jnp.maximum(_dot(p, wm_ref[...]) + bm_ref[0:1, :], 0.0)
        p = _bn(p, gm_ref[0:1, :], betam_ref[0:1, :])
    out_ref[...] = _dot(p, wo_ref[...]) + bo_ref[0:1, :]


_tc_mlp = pl.pallas_call(
    _tc_mlp_body,
    out_shape=jax.ShapeDtypeStruct((64, 1), _f32),
)


def kernel(x, edge_index, batch, W1, b1, g1, beta1, Wh, bh, gh, betah,
           Wm, bm, gm, betam, Wo, bo):
    src = edge_index[0]
    dst = edge_index[1]
    dcnt = _deg_call(dst)
    z3, dis = _tc_pre(x, W1, dcnt.reshape(2, N, 16))
    a = _agg_call(z3.reshape(2 * N, H), src, dst)
    z3 = _tc_mid_relu(a.reshape(2, N, H), dis, b1.reshape(1, DIM),
                      g1.reshape(1, DIM), beta1.reshape(1, DIM), Wh[0])
    a = _agg_call(z3.reshape(2 * N, H), src, dst)
    z3 = _tc_mid(a.reshape(2, N, H), dis, bh[0].reshape(1, DIM),
                 gh[0].reshape(1, DIM), betah[0].reshape(1, DIM), Wh[1])
    a = _agg_call(z3.reshape(2 * N, H), src, dst)
    z3 = _tc_mid(a.reshape(2, N, H), dis, bh[1].reshape(1, DIM),
                 gh[1].reshape(1, DIM), betah[1].reshape(1, DIM), Wh[2])
    a = _agg_call(z3.reshape(2 * N, H), src, dst)
    h3 = _tc_last(a.reshape(2, N, H), dis, bh[2].reshape(1, DIM),
                  gh[2].reshape(1, DIM), betah[2].reshape(1, DIM))
    p2 = _pool_call(h3.reshape(2 * N, H), batch)
    out = _tc_mlp(p2, Wm[0], Wm[1], bm[0:1], bm[1:2], gm[0:1], gm[1:2],
                  betam[0:1], betam[1:2], Wo, bo.reshape(1, 1))
    return out
